# baseline (speedup 1.0000x reference)
import functools

import jax
import jax.numpy as jnp
from jax import lax
from jax.experimental import pallas as pl
from jax.experimental.pallas import tpu as pltpu
from jax.experimental.pallas import tpu_sc as plsc

NC = 2
NS = 16
NW = NC * NS
LANE = 16

B = 16384
L = 16
SEM_D = 48
ETH_D = 16
D = 64

BPW = B // NW
CHUNK = 128
NCH = BPW // CHUNK
SCALE = 1.0 / L


def _body(idx_hbm, sem_hbm, eth_hbm, pos_hbm, out_hbm,
          idx_v, pos_v, acc_sem, acc_eth, out_c, sem_g):
    wid = lax.axis_index("s") * NC + lax.axis_index("c")
    base = wid * BPW

    pltpu.sync_copy(idx_hbm.at[pl.ds(wid * NCH, NCH)], idx_v)
    pltpu.sync_copy(pos_hbm.at[pl.ds(0, L)], pos_v)

    pos_mean = []
    for k in range(D // LANE):
        s = pos_v[0, pl.ds(k * LANE, LANE)]
        for r in range(1, L):
            s = s + pos_v[r, pl.ds(k * LANE, LANE)]
        pos_mean.append(s * SCALE)

    for c in range(NCH):
        d0 = pltpu.async_copy(sem_hbm.at[idx_v.at[c, 0]], acc_sem, sem_g)
        e0 = pltpu.async_copy(eth_hbm.at[idx_v.at[c, 0]], acc_eth, sem_g)
        d0.wait()
        e0.wait()
        descs = []
        for l in range(1, L):
            descs.append(pltpu.async_copy(
                sem_hbm.at[idx_v.at[c, l]], acc_sem, sem_g, add=True))
            descs.append(pltpu.async_copy(
                eth_hbm.at[idx_v.at[c, l]], acc_eth, sem_g, add=True))
        for dsc in descs:
            dsc.wait()

        def row_fn(r, _):
            for k in range(SEM_D // LANE):
                v = acc_sem[r, pl.ds(k * LANE, LANE)]
                out_c[r, pl.ds(k * LANE, LANE)] = v * SCALE + pos_mean[k]
            v = acc_eth[r, pl.ds(0, LANE)]
            out_c[r, pl.ds(SEM_D, LANE)] = v * SCALE + pos_mean[3]
            return 0

        lax.fori_loop(0, CHUNK, row_fn, 0)
        pltpu.sync_copy(out_c, out_hbm.at[pl.ds(base + c * CHUNK, CHUNK)])


def kernel(indices, semantic_table, ethical_table, pos_enc):
    idx_r = indices.astype(jnp.int32).reshape(B // CHUNK, CHUNK, L)
    idx_r = idx_r.transpose(0, 2, 1)
    pos2d = pos_enc.reshape(pos_enc.shape[1], D)

    run = pl.kernel(
        _body,
        out_type=jax.ShapeDtypeStruct((B, D), jnp.float32),
        mesh=plsc.VectorSubcoreMesh(core_axis_name="c", subcore_axis_name="s"),
        scratch_types=[
            pltpu.VMEM((NCH, L, CHUNK), jnp.int32),
            pltpu.VMEM((L, D), jnp.float32),
            pltpu.VMEM((CHUNK, SEM_D), jnp.float32),
            pltpu.VMEM((CHUNK, ETH_D), jnp.float32),
            pltpu.VMEM((CHUNK, D), jnp.float32),
            pltpu.SemaphoreType.DMA,
        ],
        compiler_params=pltpu.CompilerParams(use_tc_tiling_on_sc=False),
    )
    return run(idx_r, semantic_table, ethical_table, pos2d)
